# baseline (speedup 1.0000x reference)
import functools

import jax
import jax.numpy as jnp
from jax import lax
from jax.experimental import pallas as pl
from jax.experimental.pallas import tpu as pltpu
from jax.experimental.pallas import tpu_sc as plsc

F32 = jnp.float32
NC, NS = 2, 16
NW = NC * NS
BLK = 512
EK = 128


def _rup(n, m):
    return (n + m - 1) // m * m


def _pad_rows(a, n):
    return jnp.pad(a, ((0, n - a.shape[0]),) + ((0, 0),) * (a.ndim - 1))


def _pad2(a, n, k):
    return jnp.pad(a, ((0, n - a.shape[0]), (0, k - a.shape[1])))


def _silu(v):
    return v * jax.nn.sigmoid(v)


def _ln(y, g, b):
    mu = jnp.mean(y, axis=-1, keepdims=True)
    var = jnp.mean((y - mu) ** 2, axis=-1, keepdims=True)
    return (y - mu) * lax.rsqrt(var + 1e-5) * g + b


def _full_specs(ws):
    return [pl.BlockSpec(w.shape, lambda i, r=w.ndim: (0,) * r) for w in ws]


def _embed(xp, mp):
    n, kd = xp.shape
    l0, l1 = mp["layers"]
    w1 = l0["W"]
    if w1.shape[0] != kd:
        w1 = jnp.pad(w1, ((0, kd - w1.shape[0]), (0, 0)))
    ws = [w1, l0["b"][None], l1["W"], l1["b"][None], mp["ln_g"][None], mp["ln_b"][None]]

    def body(x_ref, w1r, b1r, w2r, b2r, gr, br, o_ref):
        h = _silu(jnp.dot(x_ref[...], w1r[...], preferred_element_type=F32) + b1r[...])
        y = _silu(jnp.dot(h, w2r[...], preferred_element_type=F32) + b2r[...])
        o_ref[...] = _ln(y, gr[...], br[...])

    return pl.pallas_call(
        body, grid=(n // BLK,),
        in_specs=[pl.BlockSpec((BLK, kd), lambda i: (i, 0))] + _full_specs(ws),
        out_specs=pl.BlockSpec((BLK, 128), lambda i: (i, 0)),
        out_shape=jax.ShapeDtypeStruct((n, 128), F32),
    )(xp, *ws)


def _proj(x, w):
    n, kd = x.shape
    ko = w.shape[1]

    def body(x_ref, wr, o_ref):
        o_ref[...] = jnp.dot(x_ref[...], wr[...], preferred_element_type=F32)

    return pl.pallas_call(
        body, grid=(n // BLK,),
        in_specs=[pl.BlockSpec((BLK, kd), lambda i: (i, 0))] + _full_specs([w]),
        out_specs=pl.BlockSpec((BLK, ko), lambda i: (i, 0)),
        out_shape=jax.ShapeDtypeStruct((n, ko), F32),
    )(x, w)


def _edge_update(ga, gb, eattr, w1e, b1, w2, b2, g, bb, n_real):
    n = ga.shape[0]
    ws = [w1e, b1, w2, b2, g, bb]

    def body(ga_r, gb_r, e_r, w1r, b1r, w2r, b2r, gr, br, o_ref):
        e = e_r[...]
        h = _silu(ga_r[...] + gb_r[...]
                  + jnp.dot(e, w1r[...], preferred_element_type=F32) + b1r[...])
        y = _silu(jnp.dot(h, w2r[...], preferred_element_type=F32) + b2r[...])
        out = _ln(y, gr[...], br[...]) + e
        row = pl.program_id(0) * BLK + lax.broadcasted_iota(jnp.int32, out.shape, 0)
        o_ref[...] = jnp.where(row < n_real, out, 0.0)

    return pl.pallas_call(
        body, grid=(n // BLK,),
        in_specs=[pl.BlockSpec((BLK, 128), lambda i: (i, 0))] * 3 + _full_specs(ws),
        out_specs=pl.BlockSpec((BLK, 128), lambda i: (i, 0)),
        out_shape=jax.ShapeDtypeStruct((n, 128), F32),
    )(ga, gb, eattr, *ws)


def _node_update(xd, parts, w1a, w1b, b1, w2, b2, g, bb, extra=None):
    n = xd.shape[0]
    ws = [w1a, w1b, b1, w2, b2, g, bb]
    has_extra = extra is not None

    def body(*refs):
        if has_extra:
            x_r, p_r, ex_r = refs[0], refs[1], refs[2]
            w1ar, w1br, b1r, w2r, b2r, gr, br, o_ref = refs[3:]
        else:
            x_r, p_r = refs[0], refs[1]
            w1ar, w1br, b1r, w2r, b2r, gr, br, o_ref = refs[2:]
        xv = x_r[...]
        pv = p_r[...]
        agg = pv[0] + pv[1]
        h = _silu(jnp.dot(xv, w1ar[...], preferred_element_type=F32)
                  + jnp.dot(agg, w1br[...], preferred_element_type=F32) + b1r[...])
        y = _silu(jnp.dot(h, w2r[...], preferred_element_type=F32) + b2r[...])
        out = _ln(y, gr[...], br[...]) + xv
        if has_extra:
            out = out + ex_r[...]
        o_ref[...] = out

    row = pl.BlockSpec((BLK, 128), lambda i: (i, 0))
    in_specs = [row, pl.BlockSpec((2, BLK, 128), lambda i: (0, i, 0))]
    args = [xd, parts]
    if has_extra:
        in_specs.append(row)
        args.append(extra)
    return pl.pallas_call(
        body, grid=(n // BLK,),
        in_specs=in_specs + _full_specs(ws),
        out_specs=row,
        out_shape=jax.ShapeDtypeStruct((n, 128), F32),
    )(*args, *ws)


def _extract(xv, res, mp):
    n = xv.shape[0]
    l0, l1, l2 = mp["layers"]
    out_w = l2["W"].shape[1]
    ws = [l0["W"], l0["b"][None], l1["W"], l1["b"][None], l2["W"], l2["b"][None]]

    def body(x_ref, r_ref, w1r, b1r, w2r, b2r, w3r, b3r, o_ref):
        h = _silu(jnp.dot(x_ref[...], w1r[...], preferred_element_type=F32) + b1r[...])
        h = _silu(jnp.dot(h, w2r[...], preferred_element_type=F32) + b2r[...])
        o_ref[...] = jnp.dot(h, w3r[...], preferred_element_type=F32) + b3r[...] + r_ref[...]

    return pl.pallas_call(
        body, grid=(n // BLK,),
        in_specs=[pl.BlockSpec((BLK, 128), lambda i: (i, 0)),
                  pl.BlockSpec((BLK, out_w), lambda i: (i, 0))] + _full_specs(ws),
        out_specs=pl.BlockSpec((BLK, out_w), lambda i: (i, 0)),
        out_shape=jax.ShapeDtypeStruct((n, out_w), F32),
    )(xv, res, *ws)


def _gather2(a_tab, b_tab, dst2, src2):
    _, nch_, _ = dst2.shape
    e_pad = NW * nch_ * EK
    npt = e_pad // NW
    nch = npt // EK
    mesh = plsc.VectorSubcoreMesh(core_axis_name="c", subcore_axis_name="s")

    @functools.partial(
        pl.kernel, mesh=mesh,
        out_type=[jax.ShapeDtypeStruct((e_pad, 128), F32),
                  jax.ShapeDtypeStruct((e_pad, 128), F32)],
        scratch_types=[pltpu.VMEM((nch, EK), jnp.int32), pltpu.VMEM((nch, EK), jnp.int32),
                       pltpu.VMEM((EK, 128), F32), pltpu.VMEM((EK, 128), F32),
                       pltpu.VMEM((EK, 128), F32), pltpu.VMEM((EK, 128), F32),
                       pltpu.SemaphoreType.DMA, pltpu.SemaphoreType.DMA,
                       pltpu.SemaphoreType.DMA, pltpu.SemaphoreType.DMA,
                       pltpu.SemaphoreType.DMA, pltpu.SemaphoreType.DMA,
                       pltpu.SemaphoreType.DMA, pltpu.SemaphoreType.DMA],
    )
    def kern(a_hbm, b_hbm, d_hbm, s_hbm, oa_hbm, ob_hbm,
             di, si, ra0, rb0, ra1, rb1,
             sa0, sb0, sa1, sb1, ta0, tb0, ta1, tb1):
        wid = lax.axis_index("s") * NC + lax.axis_index("c")
        base = wid * npt
        pltpu.sync_copy(d_hbm.at[wid], di)
        pltpu.sync_copy(s_hbm.at[wid], si)
        bufs = ((ra0, rb0, sa0, sb0, ta0, tb0), (ra1, rb1, sa1, sb1, ta1, tb1))

        def start(j, ra, rb, sa, sb, ta, tb):
            pltpu.async_copy(a_hbm.at[di.at[j]], ra, sa)
            pltpu.async_copy(b_hbm.at[si.at[j]], rb, sb)

        start(0, *bufs[0])

        def step(j, b):
            ra, rb, sa, sb, ta, tb = bufs[b]

            @pl.when(j + 1 < nch)
            def _():
                start(j + 1, *bufs[1 - b])

            @pl.when(j >= 2)
            def _():
                off2 = base + (j - 2) * EK
                pltpu.make_async_copy(ra, oa_hbm.at[pl.ds(off2, EK)], ta).wait()
                pltpu.make_async_copy(rb, ob_hbm.at[pl.ds(off2, EK)], tb).wait()

            pltpu.make_async_copy(a_hbm.at[di.at[j]], ra, sa).wait()
            pltpu.make_async_copy(b_hbm.at[si.at[j]], rb, sb).wait()
            off = base + j * EK
            pltpu.async_copy(ra, oa_hbm.at[pl.ds(off, EK)], ta)
            pltpu.async_copy(rb, ob_hbm.at[pl.ds(off, EK)], tb)

        def outer(j2, c):
            for b in range(2):
                step(j2 * 2 + b, b)
            return c

        lax.fori_loop(0, nch // 2, outer, 0)
        for j, b in ((nch - 2, 0), (nch - 1, 1)):
            off = base + j * EK
            ra, rb, ta, tb = bufs[b][0], bufs[b][1], bufs[b][4], bufs[b][5]
            pltpu.make_async_copy(ra, oa_hbm.at[pl.ds(off, EK)], ta).wait()
            pltpu.make_async_copy(rb, ob_hbm.at[pl.ds(off, EK)], tb).wait()

    return kern(a_tab, b_tab, dst2, src2)


def _scatter_add(vals, dst2, n_pad):
    _, nch_, _ = dst2.shape
    e_pad = NW * nch_ * EK
    npt = e_pad // NW
    nch = npt // EK
    rpt = n_pad // NS
    zeros = jnp.zeros((rpt, 128), F32)
    mesh = plsc.VectorSubcoreMesh(core_axis_name="c", subcore_axis_name="s")

    @functools.partial(
        pl.kernel, mesh=mesh,
        out_type=jax.ShapeDtypeStruct((NC * n_pad, 128), F32),
        scratch_types=[pltpu.VMEM((nch, EK), jnp.int32),
                       pltpu.VMEM((EK, 128), F32), pltpu.VMEM((EK, 128), F32),
                       pltpu.VMEM_SHARED((n_pad, 128), F32),
                       pltpu.SemaphoreType.DMA, pltpu.SemaphoreType.DMA],
    )
    def kern(v_hbm, d_hbm, z_hbm, o_hbm, di, rv0, rv1, acc, sv0, sv1):
        cid = lax.axis_index("c")
        sid = lax.axis_index("s")
        r0 = sid * rpt
        wid = cid * NS + sid
        base = wid * npt
        pltpu.sync_copy(d_hbm.at[wid], di)
        bufs = ((rv0, sv0), (rv1, sv1))

        def load(j, rv, sv):
            pltpu.async_copy(v_hbm.at[pl.ds(base + j * EK, EK)], rv, sv)

        load(0, *bufs[0])
        pltpu.sync_copy(z_hbm, acc.at[pl.ds(r0, rpt)])
        plsc.subcore_barrier()

        def outer(j2, c):
            for b in range(2):
                j = j2 * 2 + b
                rv, sv = bufs[b]

                @pl.when(j + 1 < nch)
                def _():
                    load(j + 1, *bufs[1 - b])

                pltpu.make_async_copy(v_hbm.at[pl.ds(base + j * EK, EK)], rv, sv).wait()
                pltpu.sync_copy(rv, acc.at[di.at[j]], add=True)
            return c

        lax.fori_loop(0, nch // 2, outer, 0)
        plsc.subcore_barrier()
        pltpu.sync_copy(acc.at[pl.ds(r0, rpt)], o_hbm.at[pl.ds(cid * n_pad + r0, rpt)])

    return kern(vals, dst2, zeros).reshape(NC, n_pad, 128)


def _gnn_block(bp, x_src, x_dst, src_p, dst_p, eattr, e_real, extra=None):
    em, nm = bp["edge_mlp"], bp["node_mlp"]
    eW1 = em["layers"][0]["W"]
    el1 = em["layers"][1]
    A = _proj(x_dst, eW1[:128])
    B = _proj(x_src, eW1[128:256])
    ga, gb = _gather2(A, B, dst_p, src_p)
    edges = _edge_update(ga, gb, eattr, eW1[256:], em["layers"][0]["b"][None],
                         el1["W"], el1["b"][None], em["ln_g"][None], em["ln_b"][None],
                         e_real)
    parts = _scatter_add(edges, dst_p, x_dst.shape[0])
    nW1 = nm["layers"][0]["W"]
    nl1 = nm["layers"][1]
    nodes = _node_update(x_dst, parts, nW1[:128], nW1[128:],
                         nm["layers"][0]["b"][None], nl1["W"], nl1["b"][None],
                         nm["ln_g"][None], nm["ln_b"][None], extra)
    return nodes, edges


def kernel(x, era_latlons, h_latlons, e2h_edge_index, e2h_edge_attr,
           h2h_edge_index, h2h_edge_attr, h2e_edge_index, h2e_edge_attr, params):
    p = params
    bs, n_era, _ = x.shape
    n_h = h_latlons.shape[0]
    era_pad = _rup(n_era, BLK)
    h_pad = _rup(n_h, BLK)
    out_w = p["node_era_extractor"]["layers"][-1]["W"].shape[1]

    x_flat = x.reshape(bs * n_era, -1)
    x_in = jnp.concatenate([x_flat, era_latlons], axis=-1)
    x_in = _pad2(x_in, era_pad, _rup(x_in.shape[1], 8))
    x_era = _embed(x_in, p["node_era_embedder"])
    x_h = _embed(_pad2(h_latlons, h_pad, 8), p["node_h_embedder"])

    epad = lambda e: _rup(e, NW * EK * 2)
    e2h_p = epad(e2h_edge_attr.shape[0])
    h2h_p = epad(h2h_edge_attr.shape[0])
    h2e_p = epad(h2e_edge_attr.shape[0])
    e2h_lat = _embed(_pad2(e2h_edge_attr, e2h_p, 8), p["edge_era_to_h_embedder"])
    h2h_lat = _embed(_pad2(h2h_edge_attr, h2h_p, 8), p["edge_h_to_h_embedder"])
    h2e_lat = _embed(_pad2(h2e_edge_attr, h2e_p, 8), p["edge_h_to_era_embedder"])

    pad1 = lambda v, n: jnp.pad(v, (0, n - v.shape[0])).reshape(NW, -1, EK)
    s_e2h, d_e2h = pad1(e2h_edge_index[0], e2h_p), pad1(e2h_edge_index[1], e2h_p)
    s_h2h, d_h2h = pad1(h2h_edge_index[0], h2h_p), pad1(h2h_edge_index[1], h2h_p)
    s_h2e, d_h2e = pad1(h2e_edge_index[0], h2e_p), pad1(h2e_edge_index[1], h2e_p)

    for bp in p["forward_mapper"]:
        x_h, e2h_lat = _gnn_block(bp, x_era, x_h, s_e2h, d_e2h, e2h_lat,
                                  e2h_edge_index.shape[1])
    x_latent = x_h

    x_proc = x_latent
    n_proc = len(p["h_processor"])
    for i, bp in enumerate(p["h_processor"]):
        extra = x_latent if i == n_proc - 1 else None
        x_proc, h2h_lat = _gnn_block(bp, x_proc, x_proc, s_h2h, d_h2h, h2h_lat,
                                     h2h_edge_index.shape[1], extra=extra)

    x_out = x_era
    for bp in p["backward_mapper"]:
        x_out, h2e_lat = _gnn_block(bp, x_proc, x_out, s_h2e, d_h2e, h2e_lat,
                                    h2e_edge_index.shape[1])

    res = _pad_rows(x_flat[:, :out_w], era_pad)
    y = _extract(x_out, res, p["node_era_extractor"])
    return y[:bs * n_era].reshape(bs, n_era, out_w)


# speedup vs baseline: 1.0323x; 1.0323x over previous
import functools

import jax
import jax.numpy as jnp
from jax import lax
from jax.experimental import pallas as pl
from jax.experimental.pallas import tpu as pltpu
from jax.experimental.pallas import tpu_sc as plsc

F32 = jnp.float32
NC, NS = 2, 16
NW = NC * NS
BLK = 512
EK = 128


def _rup(n, m):
    return (n + m - 1) // m * m


def _pad_rows(a, n):
    return jnp.pad(a, ((0, n - a.shape[0]),) + ((0, 0),) * (a.ndim - 1))


def _pad2(a, n, k):
    return jnp.pad(a, ((0, n - a.shape[0]), (0, k - a.shape[1])))


def _silu(v):
    return v * jax.nn.sigmoid(v)


def _ln(y, g, b):
    mu = jnp.mean(y, axis=-1, keepdims=True)
    var = jnp.mean((y - mu) ** 2, axis=-1, keepdims=True)
    return (y - mu) * lax.rsqrt(var + 1e-5) * g + b


def _full_specs(ws):
    return [pl.BlockSpec(w.shape, lambda i, r=w.ndim: (0,) * r) for w in ws]


def _embed(xp, mp):
    n, kd = xp.shape
    l0, l1 = mp["layers"]
    w1 = l0["W"]
    if w1.shape[0] != kd:
        w1 = jnp.pad(w1, ((0, kd - w1.shape[0]), (0, 0)))
    ws = [w1, l0["b"][None], l1["W"], l1["b"][None], mp["ln_g"][None], mp["ln_b"][None]]

    def body(x_ref, w1r, b1r, w2r, b2r, gr, br, o_ref):
        h = _silu(jnp.dot(x_ref[...], w1r[...], preferred_element_type=F32) + b1r[...])
        y = _silu(jnp.dot(h, w2r[...], preferred_element_type=F32) + b2r[...])
        o_ref[...] = _ln(y, gr[...], br[...])

    return pl.pallas_call(
        body, grid=(n // BLK,),
        in_specs=[pl.BlockSpec((BLK, kd), lambda i: (i, 0))] + _full_specs(ws),
        out_specs=pl.BlockSpec((BLK, 128), lambda i: (i, 0)),
        out_shape=jax.ShapeDtypeStruct((n, 128), F32),
    )(xp, *ws)


def _proj(x, w):
    n, kd = x.shape
    ko = w.shape[1]

    def body(x_ref, wr, o_ref):
        o_ref[...] = jnp.dot(x_ref[...], wr[...], preferred_element_type=F32)

    return pl.pallas_call(
        body, grid=(n // BLK,),
        in_specs=[pl.BlockSpec((BLK, kd), lambda i: (i, 0))] + _full_specs([w]),
        out_specs=pl.BlockSpec((BLK, ko), lambda i: (i, 0)),
        out_shape=jax.ShapeDtypeStruct((n, ko), F32),
    )(x, w)


def _edge_update(gs, eattr, w1e, b1, w2, b2, g, bb, n_real):
    n = gs.shape[0]
    ws = [w1e, b1, w2, b2, g, bb]

    def body(gs_r, e_r, w1r, b1r, w2r, b2r, gr, br, o_ref):
        e = e_r[...]
        h = _silu(gs_r[...]
                  + jnp.dot(e, w1r[...], preferred_element_type=F32) + b1r[...])
        y = _silu(jnp.dot(h, w2r[...], preferred_element_type=F32) + b2r[...])
        out = _ln(y, gr[...], br[...]) + e
        row = pl.program_id(0) * BLK + lax.broadcasted_iota(jnp.int32, out.shape, 0)
        o_ref[...] = jnp.where(row < n_real, out, 0.0)

    return pl.pallas_call(
        body, grid=(n // BLK,),
        in_specs=[pl.BlockSpec((BLK, 128), lambda i: (i, 0))] * 2 + _full_specs(ws),
        out_specs=pl.BlockSpec((BLK, 128), lambda i: (i, 0)),
        out_shape=jax.ShapeDtypeStruct((n, 128), F32),
    )(gs, eattr, *ws)


def _node_update(xd, parts, w1a, w1b, b1, w2, b2, g, bb, extra=None):
    n = xd.shape[0]
    ws = [w1a, w1b, b1, w2, b2, g, bb]
    has_extra = extra is not None

    def body(*refs):
        if has_extra:
            x_r, p_r, ex_r = refs[0], refs[1], refs[2]
            w1ar, w1br, b1r, w2r, b2r, gr, br, o_ref = refs[3:]
        else:
            x_r, p_r = refs[0], refs[1]
            w1ar, w1br, b1r, w2r, b2r, gr, br, o_ref = refs[2:]
        xv = x_r[...]
        pv = p_r[...]
        agg = pv[0] + pv[1]
        h = _silu(jnp.dot(xv, w1ar[...], preferred_element_type=F32)
                  + jnp.dot(agg, w1br[...], preferred_element_type=F32) + b1r[...])
        y = _silu(jnp.dot(h, w2r[...], preferred_element_type=F32) + b2r[...])
        out = _ln(y, gr[...], br[...]) + xv
        if has_extra:
            out = out + ex_r[...]
        o_ref[...] = out

    row = pl.BlockSpec((BLK, 128), lambda i: (i, 0))
    in_specs = [row, pl.BlockSpec((2, BLK, 128), lambda i: (0, i, 0))]
    args = [xd, parts]
    if has_extra:
        in_specs.append(row)
        args.append(extra)
    return pl.pallas_call(
        body, grid=(n // BLK,),
        in_specs=in_specs + _full_specs(ws),
        out_specs=row,
        out_shape=jax.ShapeDtypeStruct((n, 128), F32),
    )(*args, *ws)


def _extract(xv, res, mp):
    n = xv.shape[0]
    l0, l1, l2 = mp["layers"]
    out_w = l2["W"].shape[1]
    ws = [l0["W"], l0["b"][None], l1["W"], l1["b"][None], l2["W"], l2["b"][None]]

    def body(x_ref, r_ref, w1r, b1r, w2r, b2r, w3r, b3r, o_ref):
        h = _silu(jnp.dot(x_ref[...], w1r[...], preferred_element_type=F32) + b1r[...])
        h = _silu(jnp.dot(h, w2r[...], preferred_element_type=F32) + b2r[...])
        o_ref[...] = jnp.dot(h, w3r[...], preferred_element_type=F32) + b3r[...] + r_ref[...]

    return pl.pallas_call(
        body, grid=(n // BLK,),
        in_specs=[pl.BlockSpec((BLK, 128), lambda i: (i, 0)),
                  pl.BlockSpec((BLK, out_w), lambda i: (i, 0))] + _full_specs(ws),
        out_specs=pl.BlockSpec((BLK, out_w), lambda i: (i, 0)),
        out_shape=jax.ShapeDtypeStruct((n, out_w), F32),
    )(xv, res, *ws)


def _gather_sum(a_tab, b_tab, dst2, src2):
    _, nch_, _ = dst2.shape
    e_pad = NW * nch_ * EK
    npt = e_pad // NW
    nch = npt // EK
    mesh = plsc.VectorSubcoreMesh(core_axis_name="c", subcore_axis_name="s")

    @functools.partial(
        pl.kernel, mesh=mesh,
        out_type=jax.ShapeDtypeStruct((e_pad, 128), F32),
        scratch_types=[pltpu.VMEM((nch, EK), jnp.int32), pltpu.VMEM((nch, EK), jnp.int32),
                       pltpu.VMEM((EK, 128), F32), pltpu.VMEM((EK, 128), F32),
                       pltpu.VMEM((EK, 128), F32), pltpu.VMEM((EK, 128), F32),
                       pltpu.VMEM((EK, 128), F32), pltpu.VMEM((EK, 128), F32),
                       pltpu.SemaphoreType.DMA, pltpu.SemaphoreType.DMA,
                       pltpu.SemaphoreType.DMA, pltpu.SemaphoreType.DMA,
                       pltpu.SemaphoreType.DMA, pltpu.SemaphoreType.DMA],
    )
    def kern(a_hbm, b_hbm, d_hbm, s_hbm, o_hbm,
             di, si, ra0, rb0, ra1, rb1, rs0, rs1,
             sa0, sb0, sa1, sb1, ss0, ss1):
        wid = lax.axis_index("s") * NC + lax.axis_index("c")
        base = wid * npt
        pltpu.sync_copy(d_hbm.at[wid], di)
        pltpu.sync_copy(s_hbm.at[wid], si)
        bufs = ((ra0, rb0, rs0, sa0, sb0, ss0), (ra1, rb1, rs1, sa1, sb1, ss1))

        def start(j, ra, rb, sa, sb):
            pltpu.async_copy(a_hbm.at[di.at[j]], ra, sa)
            pltpu.async_copy(b_hbm.at[si.at[j]], rb, sb)

        start(0, ra0, rb0, sa0, sb0)

        def step(j, b):
            ra, rb, rs, sa, sb, ss = bufs[b]

            @pl.when(j + 1 < nch)
            def _():
                start(j + 1, *bufs[1 - b][:2], *bufs[1 - b][3:5])

            pltpu.make_async_copy(a_hbm.at[di.at[j]], ra, sa).wait()
            pltpu.make_async_copy(b_hbm.at[si.at[j]], rb, sb).wait()

            @pl.when(j >= 2)
            def _():
                off2 = base + (j - 2) * EK
                pltpu.make_async_copy(rs, o_hbm.at[pl.ds(off2, EK)], ss).wait()

            def addrow(i, c):
                for col in range(8):
                    sl = pl.ds(col * 16, 16)
                    rs[i, sl] = ra[i, sl] + rb[i, sl]
                return c

            lax.fori_loop(0, EK, addrow, 0)
            off = base + j * EK
            pltpu.async_copy(rs, o_hbm.at[pl.ds(off, EK)], ss)

        def outer(j2, c):
            for b in range(2):
                step(j2 * 2 + b, b)
            return c

        lax.fori_loop(0, nch // 2, outer, 0)
        for j, b in ((nch - 2, 0), (nch - 1, 1)):
            off = base + j * EK
            rs, ss = bufs[b][2], bufs[b][5]
            pltpu.make_async_copy(rs, o_hbm.at[pl.ds(off, EK)], ss).wait()

    return kern(a_tab, b_tab, dst2, src2)


def _scatter_add(vals, dst2, n_pad):
    _, nch_, _ = dst2.shape
    e_pad = NW * nch_ * EK
    npt = e_pad // NW
    nch = npt // EK
    rpt = n_pad // NS
    zeros = jnp.zeros((rpt, 128), F32)
    mesh = plsc.VectorSubcoreMesh(core_axis_name="c", subcore_axis_name="s")

    @functools.partial(
        pl.kernel, mesh=mesh,
        out_type=jax.ShapeDtypeStruct((NC * n_pad, 128), F32),
        scratch_types=[pltpu.VMEM((nch, EK), jnp.int32),
                       pltpu.VMEM((EK, 128), F32), pltpu.VMEM((EK, 128), F32),
                       pltpu.VMEM_SHARED((n_pad, 128), F32),
                       pltpu.SemaphoreType.DMA, pltpu.SemaphoreType.DMA],
    )
    def kern(v_hbm, d_hbm, z_hbm, o_hbm, di, rv0, rv1, acc, sv0, sv1):
        cid = lax.axis_index("c")
        sid = lax.axis_index("s")
        r0 = sid * rpt
        wid = cid * NS + sid
        base = wid * npt
        pltpu.sync_copy(d_hbm.at[wid], di)
        bufs = ((rv0, sv0), (rv1, sv1))

        def load(j, rv, sv):
            pltpu.async_copy(v_hbm.at[pl.ds(base + j * EK, EK)], rv, sv)

        load(0, *bufs[0])
        pltpu.sync_copy(z_hbm, acc.at[pl.ds(r0, rpt)])
        plsc.subcore_barrier()

        def outer(j2, c):
            for b in range(2):
                j = j2 * 2 + b
                rv, sv = bufs[b]

                @pl.when(j + 1 < nch)
                def _():
                    load(j + 1, *bufs[1 - b])

                pltpu.make_async_copy(v_hbm.at[pl.ds(base + j * EK, EK)], rv, sv).wait()
                pltpu.sync_copy(rv, acc.at[di.at[j]], add=True)
            return c

        lax.fori_loop(0, nch // 2, outer, 0)
        plsc.subcore_barrier()
        pltpu.sync_copy(acc.at[pl.ds(r0, rpt)], o_hbm.at[pl.ds(cid * n_pad + r0, rpt)])

    return kern(vals, dst2, zeros).reshape(NC, n_pad, 128)


def _gnn_block(bp, x_src, x_dst, src_p, dst_p, eattr, e_real, extra=None):
    em, nm = bp["edge_mlp"], bp["node_mlp"]
    eW1 = em["layers"][0]["W"]
    el1 = em["layers"][1]
    A = _proj(x_dst, eW1[:128])
    B = _proj(x_src, eW1[128:256])
    gs = _gather_sum(A, B, dst_p, src_p)
    edges = _edge_update(gs, eattr, eW1[256:], em["layers"][0]["b"][None],
                         el1["W"], el1["b"][None], em["ln_g"][None], em["ln_b"][None],
                         e_real)
    parts = _scatter_add(edges, dst_p, x_dst.shape[0])
    nW1 = nm["layers"][0]["W"]
    nl1 = nm["layers"][1]
    nodes = _node_update(x_dst, parts, nW1[:128], nW1[128:],
                         nm["layers"][0]["b"][None], nl1["W"], nl1["b"][None],
                         nm["ln_g"][None], nm["ln_b"][None], extra)
    return nodes, edges


def kernel(x, era_latlons, h_latlons, e2h_edge_index, e2h_edge_attr,
           h2h_edge_index, h2h_edge_attr, h2e_edge_index, h2e_edge_attr, params):
    p = params
    bs, n_era, _ = x.shape
    n_h = h_latlons.shape[0]
    era_pad = _rup(n_era, BLK)
    h_pad = _rup(n_h, BLK)
    out_w = p["node_era_extractor"]["layers"][-1]["W"].shape[1]

    x_flat = x.reshape(bs * n_era, -1)
    x_in = jnp.concatenate([x_flat, era_latlons], axis=-1)
    x_in = _pad2(x_in, era_pad, _rup(x_in.shape[1], 8))
    x_era = _embed(x_in, p["node_era_embedder"])
    x_h = _embed(_pad2(h_latlons, h_pad, 8), p["node_h_embedder"])

    epad = lambda e: _rup(e, NW * EK * 2)
    e2h_p = epad(e2h_edge_attr.shape[0])
    h2h_p = epad(h2h_edge_attr.shape[0])
    h2e_p = epad(h2e_edge_attr.shape[0])
    e2h_lat = _embed(_pad2(e2h_edge_attr, e2h_p, 8), p["edge_era_to_h_embedder"])
    h2h_lat = _embed(_pad2(h2h_edge_attr, h2h_p, 8), p["edge_h_to_h_embedder"])
    h2e_lat = _embed(_pad2(h2e_edge_attr, h2e_p, 8), p["edge_h_to_era_embedder"])

    pad1 = lambda v, n: jnp.pad(v, (0, n - v.shape[0])).reshape(NW, -1, EK)
    s_e2h, d_e2h = pad1(e2h_edge_index[0], e2h_p), pad1(e2h_edge_index[1], e2h_p)
    s_h2h, d_h2h = pad1(h2h_edge_index[0], h2h_p), pad1(h2h_edge_index[1], h2h_p)
    s_h2e, d_h2e = pad1(h2e_edge_index[0], h2e_p), pad1(h2e_edge_index[1], h2e_p)

    for bp in p["forward_mapper"]:
        x_h, e2h_lat = _gnn_block(bp, x_era, x_h, s_e2h, d_e2h, e2h_lat,
                                  e2h_edge_index.shape[1])
    x_latent = x_h

    x_proc = x_latent
    n_proc = len(p["h_processor"])
    for i, bp in enumerate(p["h_processor"]):
        extra = x_latent if i == n_proc - 1 else None
        x_proc, h2h_lat = _gnn_block(bp, x_proc, x_proc, s_h2h, d_h2h, h2h_lat,
                                     h2h_edge_index.shape[1], extra=extra)

    x_out = x_era
    for bp in p["backward_mapper"]:
        x_out, h2e_lat = _gnn_block(bp, x_proc, x_out, s_h2e, d_h2e, h2e_lat,
                                    h2e_edge_index.shape[1])

    res = _pad_rows(x_flat[:, :out_w], era_pad)
    y = _extract(x_out, res, p["node_era_extractor"])
    return y[:bs * n_era].reshape(bs, n_era, out_w)


# speedup vs baseline: 1.0402x; 1.0077x over previous
import functools

import jax
import jax.numpy as jnp
from jax import lax
from jax.experimental import pallas as pl
from jax.experimental.pallas import tpu as pltpu
from jax.experimental.pallas import tpu_sc as plsc

F32 = jnp.float32
NC, NS = 2, 16
NW = NC * NS
BLK = 512
EK = 128


def _rup(n, m):
    return (n + m - 1) // m * m


def _pad_rows(a, n):
    return jnp.pad(a, ((0, n - a.shape[0]),) + ((0, 0),) * (a.ndim - 1))


def _pad2(a, n, k):
    return jnp.pad(a, ((0, n - a.shape[0]), (0, k - a.shape[1])))


def _silu(v):
    return v * jax.nn.sigmoid(v)


def _ln(y, g, b):
    mu = jnp.mean(y, axis=-1, keepdims=True)
    var = jnp.mean((y - mu) ** 2, axis=-1, keepdims=True)
    return (y - mu) * lax.rsqrt(var + 1e-5) * g + b


def _full_specs(ws):
    return [pl.BlockSpec(w.shape, lambda i, r=w.ndim: (0,) * r) for w in ws]


def _embed(xp, mp):
    n, kd = xp.shape
    l0, l1 = mp["layers"]
    w1 = l0["W"]
    if w1.shape[0] != kd:
        w1 = jnp.pad(w1, ((0, kd - w1.shape[0]), (0, 0)))
    ws = [w1, l0["b"][None], l1["W"], l1["b"][None], mp["ln_g"][None], mp["ln_b"][None]]

    def body(x_ref, w1r, b1r, w2r, b2r, gr, br, o_ref):
        h = _silu(jnp.dot(x_ref[...], w1r[...], preferred_element_type=F32) + b1r[...])
        y = _silu(jnp.dot(h, w2r[...], preferred_element_type=F32) + b2r[...])
        o_ref[...] = _ln(y, gr[...], br[...])

    return pl.pallas_call(
        body, grid=(n // BLK,),
        in_specs=[pl.BlockSpec((BLK, kd), lambda i: (i, 0))] + _full_specs(ws),
        out_specs=pl.BlockSpec((BLK, 128), lambda i: (i, 0)),
        out_shape=jax.ShapeDtypeStruct((n, 128), F32),
    )(xp, *ws)


def _proj(x, w):
    n, kd = x.shape
    ko = w.shape[1]

    def body(x_ref, wr, o_ref):
        o_ref[...] = jnp.dot(x_ref[...], wr[...], preferred_element_type=F32)

    return pl.pallas_call(
        body, grid=(n // BLK,),
        in_specs=[pl.BlockSpec((BLK, kd), lambda i: (i, 0))] + _full_specs([w]),
        out_specs=pl.BlockSpec((BLK, ko), lambda i: (i, 0)),
        out_shape=jax.ShapeDtypeStruct((n, ko), F32),
    )(x, w)


def _edge_update(gs, eattr, w1e, b1, w2, b2, g, bb, n_real):
    n = gs.shape[0]
    ws = [w1e, b1, w2, b2, g, bb]

    def body(gs_r, e_r, w1r, b1r, w2r, b2r, gr, br, o_ref):
        e = e_r[...]
        h = _silu(gs_r[...]
                  + jnp.dot(e, w1r[...], preferred_element_type=F32) + b1r[...])
        y = _silu(jnp.dot(h, w2r[...], preferred_element_type=F32) + b2r[...])
        out = _ln(y, gr[...], br[...]) + e
        row = pl.program_id(0) * BLK + lax.broadcasted_iota(jnp.int32, out.shape, 0)
        o_ref[...] = jnp.where(row < n_real, out, 0.0)

    return pl.pallas_call(
        body, grid=(n // BLK,),
        in_specs=[pl.BlockSpec((BLK, 128), lambda i: (i, 0))] * 2 + _full_specs(ws),
        out_specs=pl.BlockSpec((BLK, 128), lambda i: (i, 0)),
        out_shape=jax.ShapeDtypeStruct((n, 128), F32),
    )(gs, eattr, *ws)


def _node_update(xd, parts, w1a, w1b, b1, w2, b2, g, bb, extra=None):
    n = xd.shape[0]
    ws = [w1a, w1b, b1, w2, b2, g, bb]
    has_extra = extra is not None

    def body(*refs):
        if has_extra:
            x_r, p_r, ex_r = refs[0], refs[1], refs[2]
            w1ar, w1br, b1r, w2r, b2r, gr, br, o_ref = refs[3:]
        else:
            x_r, p_r = refs[0], refs[1]
            w1ar, w1br, b1r, w2r, b2r, gr, br, o_ref = refs[2:]
        xv = x_r[...]
        pv = p_r[...]
        agg = pv[0] + pv[1]
        h = _silu(jnp.dot(xv, w1ar[...], preferred_element_type=F32)
                  + jnp.dot(agg, w1br[...], preferred_element_type=F32) + b1r[...])
        y = _silu(jnp.dot(h, w2r[...], preferred_element_type=F32) + b2r[...])
        out = _ln(y, gr[...], br[...]) + xv
        if has_extra:
            out = out + ex_r[...]
        o_ref[...] = out

    row = pl.BlockSpec((BLK, 128), lambda i: (i, 0))
    in_specs = [row, pl.BlockSpec((2, BLK, 128), lambda i: (0, i, 0))]
    args = [xd, parts]
    if has_extra:
        in_specs.append(row)
        args.append(extra)
    return pl.pallas_call(
        body, grid=(n // BLK,),
        in_specs=in_specs + _full_specs(ws),
        out_specs=row,
        out_shape=jax.ShapeDtypeStruct((n, 128), F32),
    )(*args, *ws)


def _extract(xv, res, mp):
    n = xv.shape[0]
    l0, l1, l2 = mp["layers"]
    out_w = l2["W"].shape[1]
    ws = [l0["W"], l0["b"][None], l1["W"], l1["b"][None], l2["W"], l2["b"][None]]

    def body(x_ref, r_ref, w1r, b1r, w2r, b2r, w3r, b3r, o_ref):
        h = _silu(jnp.dot(x_ref[...], w1r[...], preferred_element_type=F32) + b1r[...])
        h = _silu(jnp.dot(h, w2r[...], preferred_element_type=F32) + b2r[...])
        o_ref[...] = jnp.dot(h, w3r[...], preferred_element_type=F32) + b3r[...] + r_ref[...]

    return pl.pallas_call(
        body, grid=(n // BLK,),
        in_specs=[pl.BlockSpec((BLK, 128), lambda i: (i, 0)),
                  pl.BlockSpec((BLK, out_w), lambda i: (i, 0))] + _full_specs(ws),
        out_specs=pl.BlockSpec((BLK, out_w), lambda i: (i, 0)),
        out_shape=jax.ShapeDtypeStruct((n, out_w), F32),
    )(xv, res, *ws)


def _gather_sum(a_tab, b_tab, dst2, src2):
    _, nch_, _ = dst2.shape
    e_pad = NW * nch_ * EK
    npt = e_pad // NW
    nch = npt // EK
    mesh = plsc.VectorSubcoreMesh(core_axis_name="c", subcore_axis_name="s")

    @functools.partial(
        pl.kernel, mesh=mesh,
        out_type=jax.ShapeDtypeStruct((e_pad, 128), F32),
        scratch_types=[pltpu.VMEM((nch, EK), jnp.int32), pltpu.VMEM((nch, EK), jnp.int32),
                       pltpu.VMEM((EK, 128), F32), pltpu.VMEM((EK, 128), F32),
                       pltpu.VMEM((EK, 128), F32), pltpu.VMEM((EK, 128), F32),
                       pltpu.VMEM((EK, 128), F32), pltpu.VMEM((EK, 128), F32),
                       pltpu.SemaphoreType.DMA, pltpu.SemaphoreType.DMA,
                       pltpu.SemaphoreType.DMA, pltpu.SemaphoreType.DMA,
                       pltpu.SemaphoreType.DMA, pltpu.SemaphoreType.DMA],
    )
    def kern(a_hbm, b_hbm, d_hbm, s_hbm, o_hbm,
             di, si, ra0, rb0, ra1, rb1, rs0, rs1,
             sa0, sb0, sa1, sb1, ss0, ss1):
        wid = lax.axis_index("s") * NC + lax.axis_index("c")
        base = wid * npt
        pltpu.sync_copy(d_hbm.at[wid], di)
        pltpu.sync_copy(s_hbm.at[wid], si)
        bufs = ((ra0, rb0, rs0, sa0, sb0, ss0), (ra1, rb1, rs1, sa1, sb1, ss1))

        def start(j, ra, rb, sa, sb):
            pltpu.async_copy(a_hbm.at[di.at[j]], ra, sa)
            pltpu.async_copy(b_hbm.at[si.at[j]], rb, sb)

        start(0, ra0, rb0, sa0, sb0)

        def step(j, b):
            ra, rb, rs, sa, sb, ss = bufs[b]

            @pl.when(j + 1 < nch)
            def _():
                start(j + 1, *bufs[1 - b][:2], *bufs[1 - b][3:5])

            pltpu.make_async_copy(a_hbm.at[di.at[j]], ra, sa).wait()
            pltpu.make_async_copy(b_hbm.at[si.at[j]], rb, sb).wait()

            @pl.when(j >= 2)
            def _():
                off2 = base + (j - 2) * EK
                pltpu.make_async_copy(rs, o_hbm.at[pl.ds(off2, EK)], ss).wait()

            def addrow(i, c):
                for col in range(8):
                    sl = pl.ds(col * 16, 16)
                    rs[i, sl] = ra[i, sl] + rb[i, sl]
                return c

            lax.fori_loop(0, EK, addrow, 0)
            off = base + j * EK
            pltpu.async_copy(rs, o_hbm.at[pl.ds(off, EK)], ss)

        def outer(j2, c):
            for b in range(2):
                step(j2 * 2 + b, b)
            return c

        lax.fori_loop(0, nch // 2, outer, 0)
        for j, b in ((nch - 2, 0), (nch - 1, 1)):
            off = base + j * EK
            rs, ss = bufs[b][2], bufs[b][5]
            pltpu.make_async_copy(rs, o_hbm.at[pl.ds(off, EK)], ss).wait()

    return kern(a_tab, b_tab, dst2, src2)


def _scatter_add(vals, dst2, n_pad):
    _, nch_, _ = dst2.shape
    e_pad = NW * nch_ * EK
    npt = e_pad // NW
    nch = npt // EK
    rpt = n_pad // NS
    zeros = jnp.zeros((rpt, 128), F32)
    mesh = plsc.VectorSubcoreMesh(core_axis_name="c", subcore_axis_name="s")

    @functools.partial(
        pl.kernel, mesh=mesh,
        out_type=jax.ShapeDtypeStruct((NC * n_pad, 128), F32),
        scratch_types=[pltpu.VMEM((nch, EK), jnp.int32),
                       pltpu.VMEM((EK, 128), F32), pltpu.VMEM((EK, 128), F32),
                       pltpu.VMEM_SHARED((n_pad, 128), F32),
                       pltpu.SemaphoreType.DMA, pltpu.SemaphoreType.DMA,
                       pltpu.SemaphoreType.DMA, pltpu.SemaphoreType.DMA],
    )
    def kern(v_hbm, d_hbm, z_hbm, o_hbm, di, rv0, rv1, acc, sv0, sv1, sc0, sc1):
        cid = lax.axis_index("c")
        sid = lax.axis_index("s")
        r0 = sid * rpt
        wid = cid * NS + sid
        base = wid * npt
        pltpu.sync_copy(d_hbm.at[wid], di)
        bufs = ((rv0, sv0, sc0), (rv1, sv1, sc1))

        def load(j, rv, sv):
            pltpu.async_copy(v_hbm.at[pl.ds(base + j * EK, EK)], rv, sv)

        load(0, *bufs[0][:2])
        pltpu.sync_copy(z_hbm, acc.at[pl.ds(r0, rpt)])
        plsc.subcore_barrier()

        def outer(j2, c):
            for b in range(2):
                j = j2 * 2 + b
                rv, sv, sc = bufs[b]
                rv2, sv2, sc2 = bufs[1 - b]

                @pl.when(jnp.logical_and(j >= 1, j + 1 < nch))
                def _():
                    pltpu.make_async_copy(rv2, acc.at[di.at[j - 1]], sc2).wait()

                @pl.when(j + 1 < nch)
                def _():
                    load(j + 1, rv2, sv2)

                pltpu.make_async_copy(v_hbm.at[pl.ds(base + j * EK, EK)], rv, sv).wait()
                pltpu.async_copy(rv, acc.at[di.at[j]], sc, add=True)
            return c

        lax.fori_loop(0, nch // 2, outer, 0)
        for j, b in ((nch - 2, 0), (nch - 1, 1)):
            rv, _, sc = bufs[b]
            pltpu.make_async_copy(rv, acc.at[di.at[j]], sc).wait()
        plsc.subcore_barrier()
        pltpu.sync_copy(acc.at[pl.ds(r0, rpt)], o_hbm.at[pl.ds(cid * n_pad + r0, rpt)])

    return kern(vals, dst2, zeros).reshape(NC, n_pad, 128)


def _gnn_block(bp, x_src, x_dst, src_p, dst_p, eattr, e_real, extra=None):
    em, nm = bp["edge_mlp"], bp["node_mlp"]
    eW1 = em["layers"][0]["W"]
    el1 = em["layers"][1]
    A = _proj(x_dst, eW1[:128])
    B = _proj(x_src, eW1[128:256])
    gs = _gather_sum(A, B, dst_p, src_p)
    edges = _edge_update(gs, eattr, eW1[256:], em["layers"][0]["b"][None],
                         el1["W"], el1["b"][None], em["ln_g"][None], em["ln_b"][None],
                         e_real)
    parts = _scatter_add(edges, dst_p, x_dst.shape[0])
    nW1 = nm["layers"][0]["W"]
    nl1 = nm["layers"][1]
    nodes = _node_update(x_dst, parts, nW1[:128], nW1[128:],
                         nm["layers"][0]["b"][None], nl1["W"], nl1["b"][None],
                         nm["ln_g"][None], nm["ln_b"][None], extra)
    return nodes, edges


def kernel(x, era_latlons, h_latlons, e2h_edge_index, e2h_edge_attr,
           h2h_edge_index, h2h_edge_attr, h2e_edge_index, h2e_edge_attr, params):
    p = params
    bs, n_era, _ = x.shape
    n_h = h_latlons.shape[0]
    era_pad = _rup(n_era, BLK)
    h_pad = _rup(n_h, BLK)
    out_w = p["node_era_extractor"]["layers"][-1]["W"].shape[1]

    x_flat = x.reshape(bs * n_era, -1)
    x_in = jnp.concatenate([x_flat, era_latlons], axis=-1)
    x_in = _pad2(x_in, era_pad, _rup(x_in.shape[1], 8))
    x_era = _embed(x_in, p["node_era_embedder"])
    x_h = _embed(_pad2(h_latlons, h_pad, 8), p["node_h_embedder"])

    epad = lambda e: _rup(e, NW * EK * 2)
    e2h_p = epad(e2h_edge_attr.shape[0])
    h2h_p = epad(h2h_edge_attr.shape[0])
    h2e_p = epad(h2e_edge_attr.shape[0])
    e2h_lat = _embed(_pad2(e2h_edge_attr, e2h_p, 8), p["edge_era_to_h_embedder"])
    h2h_lat = _embed(_pad2(h2h_edge_attr, h2h_p, 8), p["edge_h_to_h_embedder"])
    h2e_lat = _embed(_pad2(h2e_edge_attr, h2e_p, 8), p["edge_h_to_era_embedder"])

    pad1 = lambda v, n: jnp.pad(v, (0, n - v.shape[0])).reshape(NW, -1, EK)
    s_e2h, d_e2h = pad1(e2h_edge_index[0], e2h_p), pad1(e2h_edge_index[1], e2h_p)
    s_h2h, d_h2h = pad1(h2h_edge_index[0], h2h_p), pad1(h2h_edge_index[1], h2h_p)
    s_h2e, d_h2e = pad1(h2e_edge_index[0], h2e_p), pad1(h2e_edge_index[1], h2e_p)

    for bp in p["forward_mapper"]:
        x_h, e2h_lat = _gnn_block(bp, x_era, x_h, s_e2h, d_e2h, e2h_lat,
                                  e2h_edge_index.shape[1])
    x_latent = x_h

    x_proc = x_latent
    n_proc = len(p["h_processor"])
    for i, bp in enumerate(p["h_processor"]):
        extra = x_latent if i == n_proc - 1 else None
        x_proc, h2h_lat = _gnn_block(bp, x_proc, x_proc, s_h2h, d_h2h, h2h_lat,
                                     h2h_edge_index.shape[1], extra=extra)

    x_out = x_era
    for bp in p["backward_mapper"]:
        x_out, h2e_lat = _gnn_block(bp, x_proc, x_out, s_h2e, d_h2e, h2e_lat,
                                    h2e_edge_index.shape[1])

    res = _pad_rows(x_flat[:, :out_w], era_pad)
    y = _extract(x_out, res, p["node_era_extractor"])
    return y[:bs * n_era].reshape(bs, n_era, out_w)
